# Initial kernel scaffold; baseline (speedup 1.0000x reference)
import jax
import jax.numpy as jnp
from jax.experimental import pallas as pl


def kernel(preds, targets):
    raise NotImplementedError("write your pallas kernel here")


# speedup vs baseline: 1.6678x; 1.6678x over previous
import functools

import jax
import jax.numpy as jnp
import numpy as np
from jax import lax
from jax.experimental import pallas as pl
from jax.experimental.pallas import tpu as pltpu
from jax.experimental.pallas import tpu_sc as plsc

B = 64
T = 20
S = 52
CH = 255
SS = S * S
NC, NS, L = 2, 16, 16
NW = NC * NS
JPW = (B // NW) * T
KP = 96
CELLS = B * T
CPC = 8
NCHUNK = JPW // CPC
RPC = CPC * KP

_AW = (10.0, 16.0, 33.0)
_AH = (13.0, 30.0, 23.0)
_IMG = 416.0
_AWN = tuple(float(np.float32(a) / np.float32(_IMG)) for a in _AW)
_AHN = tuple(float(np.float32(a) / np.float32(_IMG)) for a in _AH)
_AAH = tuple(float(np.float32(_AWN[i]) * np.float32(_AHN[i])) for i in range(3))

_LX, _LY, _LRW, _LRH, _LCLS, _LVAL = 85, 86, 87, 88, 89, 90


def _sc_body(preds_hbm, tgt_hbm, out_hbm,
             tgt_v, q_v, key_v, ext_v, idx_v, rows_v, out_v, sem):
    wid = lax.axis_index("s") * NC + lax.axis_index("c")
    iota = lax.iota(jnp.int32, L)
    f32 = jnp.float32

    pltpu.sync_copy(tgt_hbm.at[pl.ds(wid * 200, 200)], tgt_v)

    for j0 in (0, 16, 32):
        jv = iota + j0
        jc = jnp.where(jv < JPW, jv, 0)
        gx = plsc.load_gather(tgt_v, [jc * 5 + 0])
        gy = plsc.load_gather(tgt_v, [jc * 5 + 1])
        gw = plsc.load_gather(tgt_v, [jc * 5 + 2])
        gh = plsc.load_gather(tgt_v, [jc * 5 + 3])
        t4 = plsc.load_gather(tgt_v, [jc * 5 + 4])

        best = jnp.zeros((L,), jnp.int32)
        best_iou = jnp.full((L,), -1.0, f32)
        for a in range(3):
            inter = jnp.minimum(gw, f32(_AWN[a])) * jnp.minimum(gh, f32(_AHN[a]))
            union = gw * gh + f32(_AAH[a]) - inter
            iou = inter / (union + f32(1e-16))
            gt = iou > best_iou
            best = jnp.where(gt, jnp.full((L,), a, jnp.int32), best)
            best_iou = jnp.where(gt, iou, best_iou)

        gxc = gx * f32(S)
        gyc = gy * f32(S)
        cx = gxc.astype(jnp.int32)
        cy = gyc.astype(jnp.int32)
        p = cy * S + cx
        b_rel = (jv >= T).astype(jnp.int32)
        bimg = wid * 2 + b_rel
        q_v[pl.ds(j0, L)] = (bimg * CH + best * 85) * SS + p
        key_v[pl.ds(j0, L)] = best * SS + p

        aw = jnp.where(best == 0, f32(_AW[0]),
                       jnp.where(best == 1, f32(_AW[1]), f32(_AW[2])))
        ah = jnp.where(best == 0, f32(_AH[0]),
                       jnp.where(best == 1, f32(_AH[1]), f32(_AH[2])))
        ext_v[pl.ds(0 * 48 + j0, L)] = gxc - cx.astype(f32)
        ext_v[pl.ds(1 * 48 + j0, L)] = gyc - cy.astype(f32)
        ext_v[pl.ds(2 * 48 + j0, L)] = gw / aw * f32(_IMG)
        ext_v[pl.ds(3 * 48 + j0, L)] = gh / ah * f32(_IMG)
        ext_v[pl.ds(4 * 48 + j0, L)] = t4.astype(jnp.int32).astype(f32)

    koffs = [jnp.minimum(iota + g * L, 84) * SS for g in range(KP // L)]
    kmask = [(iota + g * L) < 85 for g in range(KP // L)]

    for c in range(NCHUNK):
        jbase = c * CPC

        def _fill(jj, _):
            qs = plsc.load_gather(q_v, [jnp.full((L,), jbase, jnp.int32) + jj])
            for g in range(KP // L):
                q = qs + koffs[g]
                idx_v[pl.ds(jj * KP + g * L, L)] = jnp.right_shift(q, 7)
            return 0

        lax.fori_loop(0, CPC, _fill, 0)

        copies = []
        for r in range(RPC // 128):
            copies.append(pltpu.async_copy(
                preds_hbm.at[idx_v.at[pl.ds(r * 128, 128)]],
                rows_v.at[pl.ds(r * 128, 128)], sem))
        for cp in copies:
            cp.wait()

        def _extract(jj, _):
            qs = plsc.load_gather(q_v, [jnp.full((L,), jbase, jnp.int32) + jj])
            obase = (jbase + jj) * KP
            for g in range(KP // L):
                lane = jnp.bitwise_and(qs + koffs[g], 127)
                rix = jj * KP + jnp.minimum(iota + g * L, 84)
                vals = plsc.load_gather(rows_v, [rix, lane])
                vals = jnp.where(kmask[g], vals, f32(0.0))
                out_v[pl.ds(obase + g * L, L)] = vals
            return 0

        lax.fori_loop(0, CPC, _extract, 0)

    k0 = key_v[pl.ds(0, L)]
    k1 = key_v[pl.ds(16, L)]
    k2 = key_v[pl.ds(32, L)]
    for j0 in (0, 16, 32):
        vvalid = jnp.zeros((L,), f32)
        for i in range(L):
            j = j0 + i
            if j >= JPW:
                break
            im_end = T if j < T else 2 * T
            keyt = plsc.load_gather(key_v, [jnp.full((L,), j, jnp.int32)])
            dup = jnp.zeros((L,), jnp.int32)
            for jj0, kkv in ((0, k0), (16, k1), (32, k2)):
                if jj0 + L <= j + 1 or jj0 >= im_end:
                    continue
                jjv = iota + jj0
                m = (kkv == keyt) & (jjv > j) & (jjv < im_end)
                dup = dup + plsc.all_reduce_population_count(m)
            ok = jnp.where(dup == 0, f32(1.0), f32(0.0))
            vvalid = jnp.where(iota == i, ok, vvalid)
        ext_v[pl.ds(5 * 48 + j0, L)] = vvalid

    for j0 in (0, 16, 32):
        jv = iota + j0
        maskj = jv < JPW
        xi = jv * KP
        for fi, off in ((0, _LX), (1, _LY), (2, _LRW), (3, _LRH),
                        (4, _LCLS), (5, _LVAL)):
            vec = ext_v[pl.ds(fi * 48 + j0, L)]
            plsc.store_scatter(out_v, [xi + off], vec, mask=maskj)

    pltpu.sync_copy(out_v, out_hbm.at[pl.ds(wid * (JPW * KP), JPW * KP)])


@functools.partial(
    pl.kernel,
    out_type=jax.ShapeDtypeStruct((CELLS * KP,), jnp.float32),
    mesh=plsc.VectorSubcoreMesh(core_axis_name="c", subcore_axis_name="s"),
    compiler_params=pltpu.CompilerParams(needs_layout_passes=False),
    scratch_types=[
        pltpu.VMEM((200,), jnp.float32),
        pltpu.VMEM((48,), jnp.int32),
        pltpu.VMEM((48,), jnp.int32),
        pltpu.VMEM((288,), jnp.float32),
        pltpu.VMEM((RPC,), jnp.int32),
        pltpu.VMEM((RPC, 128), jnp.float32),
        pltpu.VMEM((JPW * KP,), jnp.float32),
        pltpu.SemaphoreType.DMA,
    ],
)
def _sc_gather(preds_hbm, tgt_hbm, out_hbm, *scratch):
    _sc_body(preds_hbm, tgt_hbm, out_hbm, *scratch)


def _tc_body(obj_ref, g_ref, out_ref):
    i = pl.program_id(0)
    a = pl.program_id(1)
    step = i * 3 + a
    f32 = jnp.float32

    part = jnp.sum(jax.nn.softplus(obj_ref[...]))

    @pl.when(step == 0)
    def _init():
        out_ref[0, 0] = f32(0.0)

    out_ref[0, 0] += part

    @pl.when(step == 23)
    def _cells():
        gv = g_ref[...]
        px = gv[:, 0:1]
        py = gv[:, 1:2]
        pw = gv[:, 2:3]
        ph = gv[:, 3:4]
        pobj = gv[:, 4:5]
        x_t = gv[:, _LX:_LX + 1]
        y_t = gv[:, _LY:_LY + 1]
        rw = gv[:, _LRW:_LRW + 1]
        rh = gv[:, _LRH:_LRH + 1]
        cls = gv[:, _LCLS:_LCLS + 1].astype(jnp.int32)
        valid = gv[:, _LVAL:_LVAL + 1]

        li = lax.broadcasted_iota(jnp.int32, (CELLS, KP), 1)
        mch = (li >= 5) & (li < 85)
        neg = jnp.full(gv.shape, -jnp.inf, f32)
        logits = jnp.where(mch, gv, neg)
        m = jnp.max(logits, axis=1, keepdims=True)
        e = jnp.where(mch, jnp.exp(gv - m), f32(0.0))
        logz = jnp.log(jnp.sum(e, axis=1, keepdims=True)) + m
        picked = jnp.sum(jnp.where(li == cls + 5, gv, f32(0.0)),
                         axis=1, keepdims=True)

        xs = jax.nn.sigmoid(px)
        ys = jax.nn.sigmoid(py)
        w_t = jnp.log(rw + f32(1e-16))
        h_t = jnp.log(rh + f32(1e-16))
        cell = (-pobj
                + (xs - x_t) ** 2 + (ys - y_t) ** 2
                + (pw - w_t) ** 2 + (ph - h_t) ** 2
                + (logz - picked))
        out_ref[0, 0] = (out_ref[0, 0] + jnp.sum(valid * cell)) * f32(1.0 / B)


def kernel(preds, targets):
    preds_flat = preds.reshape(B * CH * SS // 128, 128)
    tgt_flat = targets.reshape(B * T * 5)
    gath = _sc_gather(preds_flat, tgt_flat)
    gath2d = gath.reshape(CELLS, KP)

    res = pl.pallas_call(
        _tc_body,
        grid=(8, 3),
        in_specs=[
            pl.BlockSpec((8, 1, S, S), lambda i, a: (i, 4 + 85 * a, 0, 0)),
            pl.BlockSpec((CELLS, KP), lambda i, a: (0, 0)),
        ],
        out_specs=pl.BlockSpec((1, 1), lambda i, a: (0, 0),
                               memory_space=pltpu.SMEM),
        out_shape=jax.ShapeDtypeStruct((1, 1), jnp.float32),
    )(preds, gath2d)
    return res[0, 0]
